# baseline (speedup 1.0000x reference)
import functools

import jax
import jax.numpy as jnp
from jax import lax
from jax.experimental import pallas as pl
from jax.experimental.pallas import tpu as pltpu
from jax.experimental.pallas import tpu_sc as plsc

NUM_SPEAKERS = 100
EMB = 512
BATCH = 16384

_info = plsc.get_sparse_core_info()
_NC, _NS = _info.num_cores, _info.num_subcores
NW = _NC * _NS
B_PER_W = BATCH // NW
CH = 32
NCHUNK = B_PER_W // CH
NBUF = 4
LANES = 16
VPR = EMB // LANES


@functools.partial(
    pl.kernel,
    mesh=plsc.VectorSubcoreMesh(core_axis_name="c", subcore_axis_name="s"),
    out_type=jax.ShapeDtypeStruct((BATCH, EMB), jnp.float32),
    compiler_params=pltpu.CompilerParams(needs_layout_passes=False),
    scratch_types=[
        pltpu.VMEM((B_PER_W,), jnp.int32),
        pltpu.VMEM((NUM_SPEAKERS * EMB,), jnp.float32),
        pltpu.VMEM((NBUF, CH, EMB), jnp.float32),
        pltpu.SemaphoreType.DMA,
        pltpu.SemaphoreType.DMA,
        pltpu.SemaphoreType.DMA,
        pltpu.SemaphoreType.DMA,
        pltpu.SemaphoreType.DMA,
    ],
)
def _sc_lookup(idx_hbm, table_hbm, out_hbm, idx_v, table_v, rows_v,
               sw0, sw1, sw2, sw3, s_in):
    wid = lax.axis_index("s") * _NC + lax.axis_index("c")
    base = wid * B_PER_W
    cp_idx = pltpu.async_copy(idx_hbm.at[pl.ds(base, B_PER_W)], idx_v, s_in)
    pltpu.sync_copy(table_hbm, table_v)
    cp_idx.wait()

    sw = (sw0, sw1, sw2, sw3)

    def fill_chunk(j, b):
        half = VPR // 2

        LAG = 8

        def group(g, _):
            iv = idx_v[pl.ds(j * CH + g * LANES, LANES)]

            def bcast(k):
                rk = jnp.take_along_axis(
                    iv, jnp.full((LANES,), k, jnp.int32), axis=0)
                return rk * EMB + lax.iota(jnp.int32, LANES)

            def load(rb, c):
                view = table_v.at[pl.ds(c * LANES, NUM_SPEAKERS * EMB - c * LANES)]
                return plsc.load_gather(view, [rb])

            rk = {0: bcast(0)}
            pend = []
            for k in range(LANES):
                if k + 1 < LANES:
                    rk[k + 1] = bcast(k + 1)
                for c in range(VPR):
                    pend.append((k, c, load(rk[k], c)))
                    if len(pend) > LAG:
                        kk, cc, vv = pend.pop(0)
                        rows_v[b, g * LANES + kk,
                               pl.ds(cc * LANES, LANES)] = vv
            for kk, cc, vv in pend:
                rows_v[b, g * LANES + kk, pl.ds(cc * LANES, LANES)] = vv
            return 0

        lax.fori_loop(0, CH // LANES, group, 0, unroll=False)

    def drain(b):
        pltpu.make_async_copy(
            rows_v.at[b], out_hbm.at[pl.ds(0, CH)], sw[b]).wait()

    def pair(p, _):
        for b in range(NBUF):
            j = p * NBUF + b

            @pl.when(p > 0)
            def _():
                drain(b)

            fill_chunk(j, b)
            pltpu.async_copy(
                rows_v.at[b], out_hbm.at[pl.ds(base + j * CH, CH)], sw[b])
        return 0

    lax.fori_loop(0, NCHUNK // NBUF, pair, 0, unroll=False)
    for b in range(NBUF):
        drain(b)


def kernel(speaker_id, table):
    return _sc_lookup(speaker_id.astype(jnp.int32), table.reshape(-1))


# speedup vs baseline: 1.2315x; 1.2315x over previous
import functools

import jax
import jax.numpy as jnp
from jax import lax
from jax.experimental import pallas as pl
from jax.experimental.pallas import tpu as pltpu
from jax.experimental.pallas import tpu_sc as plsc

NUM_SPEAKERS = 100
EMB = 512
BATCH = 16384

_info = plsc.get_sparse_core_info()
_NC, _NS = _info.num_cores, _info.num_subcores
NW = _NC * _NS
B_PER_W = BATCH // NW
CH = 64
NCHUNK = B_PER_W // CH
NBUF = 2
LANES = 16
VPR = EMB // LANES


@functools.partial(
    pl.kernel,
    mesh=plsc.VectorSubcoreMesh(core_axis_name="c", subcore_axis_name="s"),
    out_type=jax.ShapeDtypeStruct((BATCH, EMB), jnp.float32),
    compiler_params=pltpu.CompilerParams(needs_layout_passes=False),
    scratch_types=[
        pltpu.VMEM((B_PER_W,), jnp.int32),
        pltpu.VMEM((NUM_SPEAKERS * EMB,), jnp.float32),
        pltpu.VMEM((NBUF, CH, EMB), jnp.float32),
        pltpu.SemaphoreType.DMA,
        pltpu.SemaphoreType.DMA,
        pltpu.SemaphoreType.DMA,
    ],
)
def _sc_lookup(idx_hbm, table_hbm, out_hbm, idx_v, table_v, rows_v,
               sw0, sw1, s_in):
    wid = lax.axis_index("s") * _NC + lax.axis_index("c")
    base = wid * B_PER_W
    cp_idx = pltpu.async_copy(idx_hbm.at[pl.ds(base, B_PER_W)], idx_v, s_in)
    pltpu.sync_copy(table_hbm, table_v)
    cp_idx.wait()

    sw = (sw0, sw1)

    def fill_chunk(j, b):
        half = VPR // 2

        LAG = 8

        def group(g, _):
            iv = idx_v[pl.ds(j * CH + g * LANES, LANES)]

            def bcast(k):
                rk = jnp.take_along_axis(
                    iv, jnp.full((LANES,), k, jnp.int32), axis=0)
                return rk * EMB + lax.iota(jnp.int32, LANES)

            def load(rb, c):
                view = table_v.at[pl.ds(c * LANES, NUM_SPEAKERS * EMB - c * LANES)]
                return plsc.load_gather(view, [rb])

            rk = {0: bcast(0)}
            pend = []
            for k in range(LANES):
                if k + 1 < LANES:
                    rk[k + 1] = bcast(k + 1)
                for c in range(VPR):
                    pend.append((k, c, load(rk[k], c)))
                    if len(pend) > LAG:
                        kk, cc, vv = pend.pop(0)
                        rows_v[b, g * LANES + kk,
                               pl.ds(cc * LANES, LANES)] = vv
            for kk, cc, vv in pend:
                rows_v[b, g * LANES + kk, pl.ds(cc * LANES, LANES)] = vv
            return 0

        lax.fori_loop(0, CH // LANES, group, 0, unroll=False)

    def drain(b):
        pltpu.make_async_copy(
            rows_v.at[b], out_hbm.at[pl.ds(0, CH)], sw[b]).wait()

    def pair(p, _):
        for b in range(NBUF):
            j = p * NBUF + b

            @pl.when(p > 0)
            def _():
                drain(b)

            fill_chunk(j, b)
            pltpu.async_copy(
                rows_v.at[b], out_hbm.at[pl.ds(base + j * CH, CH)], sw[b])
        return 0

    lax.fori_loop(0, NCHUNK // NBUF, pair, 0, unroll=False)
    for b in range(NBUF):
        drain(b)


def kernel(speaker_id, table):
    return _sc_lookup(speaker_id.astype(jnp.int32), table.reshape(-1))
